# baseline (speedup 1.0000x reference)
import jax
import jax.numpy as jnp
from jax import lax
from jax.experimental import pallas as pl
from jax.experimental.pallas import tpu as pltpu
from jax.experimental.pallas import tpu_sc as plsc

_N = 10000
_E = 320000
_G = 64
_NC = 2
_NS = 16
_NW = _NC * _NS
_L = 16

_NPAD = 10240
_PT = _NPAD // _NS
_SL = _NPAD // _NW

_EBLK = _E // 128
_EB_W = _EBLK // _NW
_ER_W = _EBLK - _EB_W * _NW
_EPT = _EB_W * 128
_EB_S = _EBLK // _NS
_ER_S = _EBLK - _EB_S * _NS
_EPS = _EB_S * 128

_mesh = plsc.VectorSubcoreMesh(
    core_axis_name="c", subcore_axis_name="s", num_cores=_NC, num_subcores=_NS
)

_sc_params = pltpu.CompilerParams(
    needs_layout_passes=False, disable_bounds_checks=True)


def _rsqrt16(x):
    i = plsc.bitcast(x, jnp.int32)
    y = plsc.bitcast(jnp.int32(0x5F3759DF) - (i >> 1), jnp.float32)
    for _ in range(3):
        y = y * (1.5 - 0.5 * x * y * y)
    return y


def _h_body(x_ref, w_ref, h_ref):
    h_ref[...] = lax.dot_general(
        x_ref[...], w_ref[...],
        dimension_numbers=(((1,), (1,)), ((), ())),
        preferred_element_type=jnp.float32,
    )


def _h_matvec(x, W1):
    blk = 1000
    return pl.pallas_call(
        _h_body,
        grid=(_N // blk,),
        in_specs=[
            pl.BlockSpec((blk, 128), lambda i: (i, 0)),
            pl.BlockSpec((1, 128), lambda i: (0, 0)),
        ],
        out_specs=pl.BlockSpec((blk, 1), lambda i: (i, 0)),
        out_shape=jax.ShapeDtypeStruct((_N, 1), jnp.float32),
    )(x, W1)


def _main_body(ei_hbm, h_hbm, batch_hbm,
               out_hbm, hists_hbm, s_hbm, c_hbm,
               ec_v, ch2_v, hist, hb_v,
               h_sl, b_sl, s_sl, c_sl, s_tab, c_tab, bins,
               sem_r, sem_c, sem_h):
    c = lax.axis_index("c")
    s = lax.axis_index("s")
    wid = c * _NS + s

    eb = pl.multiple_of((_EB_W * wid + jnp.minimum(wid, _ER_W)) * 128, 128)
    cp_e = pltpu.async_copy(
        ei_hbm.at[:, pl.ds(eb, _EPT)], ec_v.at[:, pl.ds(0, _EPT)], sem_r)
    eb2 = pl.multiple_of(
        eb + jnp.where(wid < _ER_W, _EPT, 0).astype(jnp.int32), 128)
    cp_e2 = pltpu.async_copy(
        ei_hbm.at[:, pl.ds(eb2, 128)], ec_v.at[:, pl.ds(_EPT, 128)], sem_r)
    hb = pl.multiple_of((_EB_S * s + jnp.minimum(s, _ER_S)) * 128, 128)
    cp_h = pltpu.async_copy(
        ei_hbm.at[:, pl.ds(hb, _EPS)], ch2_v.at[:, pl.ds(0, _EPS)], sem_h)
    hb2 = pl.multiple_of(
        hb + jnp.where(s < _ER_S, _EPS, 0).astype(jnp.int32), 128)
    cp_h2 = pltpu.async_copy(
        ei_hbm.at[:, pl.ds(hb2, 128)], ch2_v.at[:, pl.ds(_EPS, 128)], sem_h)

    nb = pl.multiple_of(s * _PT, _PT)
    with jax.named_scope("ph_stage"):
        pltpu.sync_copy(h_hbm.at[pl.ds(nb, _PT)], h_sl)
        pltpu.sync_copy(batch_hbm.at[pl.ds(nb, _PT)], b_sl)

        zi16 = jnp.zeros((_L,), jnp.int32)

        @pl.loop(0, _NPAD // _L, unroll=8)
        def _(k):
            hist[pl.ds(k * _L, _L)] = zi16

        cp_h.wait()
        cp_h2.wait()

    onesi = jnp.full((_L,), 1, jnp.int32)

    with jax.named_scope("ph_hist"):
        @pl.loop(0, _EPS // _L, unroll=8)
        def _(j):
            idx = ch2_v[1, pl.ds(j * _L, _L)]
            plsc.addupdate_scatter(hist, [idx], onesi)

        @pl.when(s < _ER_S)
        def _():
            @pl.loop(0, 128 // _L)
            def _(j):
                idx = ch2_v[1, pl.ds(_EPS + j * _L, _L)]
                plsc.addupdate_scatter(hist, [idx], onesi)

    with jax.named_scope("ph_hpub"):
        pltpu.sync_copy(
            hist, hists_hbm.at[pl.ds(pl.multiple_of(wid * _NPAD, _NPAD), _NPAD)])
        plsc.subcore_barrier()

        cps = []
        for t in range(_NS):
            src = pl.ds(pl.multiple_of((c * _NS + t) * _NPAD, _NPAD) + nb, _PT)
            cps.append(pltpu.async_copy(hists_hbm.at[src], hb_v.at[t], sem_h))
        for cp in cps:
            cp.wait()

    with jax.named_scope("ph_tab"):
        @pl.loop(0, _PT // _L)
        def _(k):
            sl = pl.ds(k * _L, _L)
            acc = hb_v[0, sl]
            for t in range(1, _NS):
                acc = acc + hb_v[t, sl]
            deg = acc.astype(jnp.float32) + 1.0
            y = _rsqrt16(deg)
            s_sl[sl] = h_sl[sl] * y
            c_sl[sl] = ((plsc.bitcast(y, jnp.int32) & jnp.int32(-1024))
                        | (b_sl[sl] << 4))

        pltpu.sync_copy(s_sl, s_hbm.at[pl.ds(nb, _PT)])
        pltpu.sync_copy(c_sl, c_hbm.at[pl.ds(nb, _PT)])
        plsc.subcore_barrier()

    with jax.named_scope("ph_fan"):
        cp_s = pltpu.async_copy(s_hbm, s_tab, sem_r)
        cp_t = pltpu.async_copy(c_hbm, c_tab, sem_c)

        z16 = jnp.zeros((_L,), jnp.float32)

        @pl.loop(0, _G)
        def _(g):
            bins[pl.ds(g * _L, _L)] = z16

        cp_e.wait()
        cp_e2.wait()
        cp_s.wait()
        cp_t.wait()

    lane = lax.iota(jnp.int32, _L)
    mhi = jnp.int32(-1024)
    mlo = jnp.int32(1008)

    with jax.named_scope("ph_edge"):
        @pl.loop(0, _EPT // _L, unroll=8)
        def _(j):
            sl = pl.ds(j * _L, _L)
            r = ec_v[0, sl]
            cc = ec_v[1, sl]
            sv = plsc.load_gather(s_tab, [r])
            cv = plsc.load_gather(c_tab, [cc])
            dv = plsc.bitcast(cv & mhi, jnp.float32)
            plsc.addupdate_scatter(bins, [(cv & mlo) + lane], sv * dv)

        @pl.when(wid < _ER_W)
        def _():
            @pl.loop(0, 128 // _L)
            def _(j):
                sl = pl.ds(_EPT + j * _L, _L)
                r = ec_v[0, sl]
                cc = ec_v[1, sl]
                sv = plsc.load_gather(s_tab, [r])
                cv = plsc.load_gather(c_tab, [cc])
                dv = plsc.bitcast(cv & mhi, jnp.float32)
                plsc.addupdate_scatter(bins, [(cv & mlo) + lane], sv * dv)

    sb = pl.multiple_of(wid * _SL, _SL)

    with jax.named_scope("ph_self"):
        @pl.loop(0, _SL // _L, unroll=4)
        def _(k):
            sl = pl.ds(sb + k * _L, _L)
            cv = c_tab[sl]
            val = s_tab[sl] * plsc.bitcast(cv & mhi, jnp.float32)
            plsc.addupdate_scatter(bins, [(cv & mlo) + lane], val)

        pltpu.sync_copy(
            bins,
            out_hbm.at[pl.ds(pl.multiple_of(wid * _G * _L, _G * _L), _G * _L)])


def _main_call(edge_index, h_pad, batch_pad):
    kern = pl.kernel(
        _main_body,
        out_type=(
            jax.ShapeDtypeStruct((_NW * _G * _L,), jnp.float32),
            jax.ShapeDtypeStruct((_NW * _NPAD,), jnp.int32),
            jax.ShapeDtypeStruct((_NPAD,), jnp.float32),
            jax.ShapeDtypeStruct((_NPAD,), jnp.int32),
        ),
        mesh=_mesh,
        scratch_types=[
            pltpu.VMEM((2, _EPT + 128), jnp.int32),
            pltpu.VMEM((2, _EPS + 128), jnp.int32),
            pltpu.VMEM((_NPAD,), jnp.int32),
            pltpu.VMEM((_NS, _PT), jnp.int32),
            pltpu.VMEM((_PT,), jnp.float32),
            pltpu.VMEM((_PT,), jnp.int32),
            pltpu.VMEM((_PT,), jnp.float32),
            pltpu.VMEM((_PT,), jnp.int32),
            pltpu.VMEM((_NPAD,), jnp.float32),
            pltpu.VMEM((_NPAD,), jnp.int32),
            pltpu.VMEM((_G * _L,), jnp.float32),
            pltpu.SemaphoreType.DMA,
            pltpu.SemaphoreType.DMA,
            pltpu.SemaphoreType.DMA,
        ],
        compiler_params=_sc_params,
    )
    bins, _, _, _ = kern(edge_index, h_pad, batch_pad)
    return bins


def _final_body(b_ref, wp_ref, bp_ref, o_ref):
    t = jnp.sum(b_ref[...], axis=(0, 2))[:, None]
    o_ref[...] = t * wp_ref[...] + bp_ref[...][None, :]


def _final_call(bins3d, Wp, bp):
    return pl.pallas_call(
        _final_body,
        out_shape=jax.ShapeDtypeStruct((_G, 1), jnp.float32),
    )(bins3d, Wp, bp)


def kernel(x, edge_index, batch, W1, Wp, bp):
    h = _h_matvec(x, W1)
    h_pad = jnp.pad(h[:, 0], (0, _NPAD - _N))
    batch_pad = jnp.pad(batch, (0, _NPAD - _N))
    bins = _main_call(edge_index, h_pad, batch_pad)
    return _final_call(bins.reshape(_NW, _G, _L), Wp, bp)


# speedup vs baseline: 1.2046x; 1.2046x over previous
import jax
import jax.numpy as jnp
from jax import lax
from jax.experimental import pallas as pl
from jax.experimental.pallas import tpu as pltpu
from jax.experimental.pallas import tpu_sc as plsc

_N = 10000
_E = 320000
_G = 64
_NC = 2
_NS = 16
_NW = _NC * _NS
_L = 16

_NPAD = 10240
_PT = _NPAD // _NS
_SL = _NPAD // _NW

_EBLK = _E // 128
_EB_W = _EBLK // _NW
_ER_W = _EBLK - _EB_W * _NW
_EPT = _EB_W * 128
_EB_S = _EBLK // _NS
_ER_S = _EBLK - _EB_S * _NS
_EPS = _EB_S * 128

_mesh = plsc.VectorSubcoreMesh(
    core_axis_name="c", subcore_axis_name="s", num_cores=_NC, num_subcores=_NS
)

_sc_params = pltpu.CompilerParams(
    needs_layout_passes=False, disable_bounds_checks=True)


def _rsqrt16(x):
    i = plsc.bitcast(x, jnp.int32)
    y = plsc.bitcast(jnp.int32(0x5F3759DF) - (i >> 1), jnp.float32)
    for _ in range(3):
        y = y * (1.5 - 0.5 * x * y * y)
    return y


def _h_body(w_ref, x_ref, b_ref, h_ref, bp_ref):
    hv = lax.dot_general(
        w_ref[...], x_ref[...],
        dimension_numbers=(((1,), (1,)), ((), ())),
        preferred_element_type=jnp.float32,
    )
    h_ref[pl.ds(0, _N)] = hv[0, :]
    h_ref[pl.ds(_N, _NPAD - _N)] = jnp.zeros((_NPAD - _N,), jnp.float32)
    bp_ref[pl.ds(0, _N)] = b_ref[...]
    bp_ref[pl.ds(_N, _NPAD - _N)] = jnp.zeros((_NPAD - _N,), jnp.int32)


def _h_matvec(x, W1, batch):
    return pl.pallas_call(
        _h_body,
        out_shape=(
            jax.ShapeDtypeStruct((_NPAD,), jnp.float32),
            jax.ShapeDtypeStruct((_NPAD,), jnp.int32),
        ),
    )(W1, x, batch)


def _main_body(ei_hbm, h_hbm, batch_hbm,
               out_hbm, hists_hbm, s_hbm, c_hbm,
               ec_v, ch2_v, hist, hb_v,
               h_sl, b_sl, s_sl, c_sl, s_tab, c_tab, bins,
               sem_r, sem_c, sem_h):
    c = lax.axis_index("c")
    s = lax.axis_index("s")
    wid = c * _NS + s

    eb = pl.multiple_of((_EB_W * wid + jnp.minimum(wid, _ER_W)) * 128, 128)
    cp_e = pltpu.async_copy(
        ei_hbm.at[:, pl.ds(eb, _EPT)], ec_v.at[:, pl.ds(0, _EPT)], sem_r)
    eb2 = pl.multiple_of(
        eb + jnp.where(wid < _ER_W, _EPT, 0).astype(jnp.int32), 128)
    cp_e2 = pltpu.async_copy(
        ei_hbm.at[:, pl.ds(eb2, 128)], ec_v.at[:, pl.ds(_EPT, 128)], sem_r)
    hb = pl.multiple_of((_EB_S * s + jnp.minimum(s, _ER_S)) * 128, 128)
    cp_h = pltpu.async_copy(
        ei_hbm.at[:, pl.ds(hb, _EPS)], ch2_v.at[:, pl.ds(0, _EPS)], sem_h)
    hb2 = pl.multiple_of(
        hb + jnp.where(s < _ER_S, _EPS, 0).astype(jnp.int32), 128)
    cp_h2 = pltpu.async_copy(
        ei_hbm.at[:, pl.ds(hb2, 128)], ch2_v.at[:, pl.ds(_EPS, 128)], sem_h)

    nb = pl.multiple_of(s * _PT, _PT)
    with jax.named_scope("ph_stage"):
        pltpu.sync_copy(h_hbm.at[pl.ds(nb, _PT)], h_sl)
        pltpu.sync_copy(batch_hbm.at[pl.ds(nb, _PT)], b_sl)

        zi16 = jnp.zeros((_L,), jnp.int32)

        @pl.loop(0, _NPAD // _L, unroll=8)
        def _(k):
            hist[pl.ds(k * _L, _L)] = zi16

        cp_h.wait()
        cp_h2.wait()

    onesi = jnp.full((_L,), 1, jnp.int32)

    with jax.named_scope("ph_hist"):
        @pl.loop(0, _EPS // _L, unroll=8)
        def _(j):
            idx = ch2_v[1, pl.ds(j * _L, _L)]
            plsc.addupdate_scatter(hist, [idx], onesi)

        @pl.when(s < _ER_S)
        def _():
            @pl.loop(0, 128 // _L)
            def _(j):
                idx = ch2_v[1, pl.ds(_EPS + j * _L, _L)]
                plsc.addupdate_scatter(hist, [idx], onesi)

    with jax.named_scope("ph_hpub"):
        pltpu.sync_copy(
            hist, hists_hbm.at[pl.ds(pl.multiple_of(wid * _NPAD, _NPAD), _NPAD)])
        plsc.subcore_barrier()

        cps = []
        for t in range(_NS):
            src = pl.ds(pl.multiple_of((c * _NS + t) * _NPAD, _NPAD) + nb, _PT)
            cps.append(pltpu.async_copy(hists_hbm.at[src], hb_v.at[t], sem_h))
        for cp in cps:
            cp.wait()

    with jax.named_scope("ph_tab"):
        @pl.loop(0, _PT // _L)
        def _(k):
            sl = pl.ds(k * _L, _L)
            acc = hb_v[0, sl]
            for t in range(1, _NS):
                acc = acc + hb_v[t, sl]
            deg = acc.astype(jnp.float32) + 1.0
            y = _rsqrt16(deg)
            s_sl[sl] = h_sl[sl] * y
            c_sl[sl] = ((plsc.bitcast(y, jnp.int32) & jnp.int32(-1024))
                        | (b_sl[sl] << 4))

        pltpu.sync_copy(s_sl, s_hbm.at[pl.ds(nb, _PT)])
        pltpu.sync_copy(c_sl, c_hbm.at[pl.ds(nb, _PT)])
        plsc.subcore_barrier()

    with jax.named_scope("ph_fan"):
        cp_s = pltpu.async_copy(s_hbm, s_tab, sem_r)
        cp_t = pltpu.async_copy(c_hbm, c_tab, sem_c)

        z16 = jnp.zeros((_L,), jnp.float32)

        @pl.loop(0, _G)
        def _(g):
            bins[pl.ds(g * _L, _L)] = z16

        cp_e.wait()
        cp_e2.wait()
        cp_s.wait()
        cp_t.wait()

    lane = lax.iota(jnp.int32, _L)
    mhi = jnp.int32(-1024)
    mlo = jnp.int32(1008)

    with jax.named_scope("ph_edge"):
        @pl.loop(0, _EPT // _L, unroll=8)
        def _(j):
            sl = pl.ds(j * _L, _L)
            r = ec_v[0, sl]
            cc = ec_v[1, sl]
            sv = plsc.load_gather(s_tab, [r])
            cv = plsc.load_gather(c_tab, [cc])
            dv = plsc.bitcast(cv & mhi, jnp.float32)
            plsc.addupdate_scatter(bins, [(cv & mlo) + lane], sv * dv)

        @pl.when(wid < _ER_W)
        def _():
            @pl.loop(0, 128 // _L)
            def _(j):
                sl = pl.ds(_EPT + j * _L, _L)
                r = ec_v[0, sl]
                cc = ec_v[1, sl]
                sv = plsc.load_gather(s_tab, [r])
                cv = plsc.load_gather(c_tab, [cc])
                dv = plsc.bitcast(cv & mhi, jnp.float32)
                plsc.addupdate_scatter(bins, [(cv & mlo) + lane], sv * dv)

    sb = pl.multiple_of(wid * _SL, _SL)

    with jax.named_scope("ph_self"):
        @pl.loop(0, _SL // _L, unroll=4)
        def _(k):
            sl = pl.ds(sb + k * _L, _L)
            cv = c_tab[sl]
            val = s_tab[sl] * plsc.bitcast(cv & mhi, jnp.float32)
            plsc.addupdate_scatter(bins, [(cv & mlo) + lane], val)

        pltpu.sync_copy(
            bins,
            out_hbm.at[pl.ds(pl.multiple_of(wid * _G * _L, _G * _L), _G * _L)])


def _main_call(edge_index, h_pad, batch_pad):
    kern = pl.kernel(
        _main_body,
        out_type=(
            jax.ShapeDtypeStruct((_NW * _G * _L,), jnp.float32),
            jax.ShapeDtypeStruct((_NW * _NPAD,), jnp.int32),
            jax.ShapeDtypeStruct((_NPAD,), jnp.float32),
            jax.ShapeDtypeStruct((_NPAD,), jnp.int32),
        ),
        mesh=_mesh,
        scratch_types=[
            pltpu.VMEM((2, _EPT + 128), jnp.int32),
            pltpu.VMEM((2, _EPS + 128), jnp.int32),
            pltpu.VMEM((_NPAD,), jnp.int32),
            pltpu.VMEM((_NS, _PT), jnp.int32),
            pltpu.VMEM((_PT,), jnp.float32),
            pltpu.VMEM((_PT,), jnp.int32),
            pltpu.VMEM((_PT,), jnp.float32),
            pltpu.VMEM((_PT,), jnp.int32),
            pltpu.VMEM((_NPAD,), jnp.float32),
            pltpu.VMEM((_NPAD,), jnp.int32),
            pltpu.VMEM((_G * _L,), jnp.float32),
            pltpu.SemaphoreType.DMA,
            pltpu.SemaphoreType.DMA,
            pltpu.SemaphoreType.DMA,
        ],
        compiler_params=_sc_params,
    )
    bins, _, _, _ = kern(edge_index, h_pad, batch_pad)
    return bins


def _final_body(b_ref, wp_ref, bp_ref, o_ref):
    t = jnp.sum(b_ref[...], axis=(0, 2))[:, None]
    o_ref[...] = t * wp_ref[...] + bp_ref[...][None, :]


def _final_call(bins3d, Wp, bp):
    return pl.pallas_call(
        _final_body,
        out_shape=jax.ShapeDtypeStruct((_G, 1), jnp.float32),
    )(bins3d, Wp, bp)


def kernel(x, edge_index, batch, W1, Wp, bp):
    h_pad, batch_pad = _h_matvec(x, W1, batch)
    bins = _main_call(edge_index, h_pad, batch_pad)
    return _final_call(bins.reshape(_NW, _G, _L), Wp, bp)
